# Initial kernel scaffold; baseline (speedup 1.0000x reference)
import jax
import jax.numpy as jnp
from jax.experimental import pallas as pl


def kernel(node_attrs, node_feats, edge_attrs, edge_feats, edge_index, W_tpw, W_lin1, W_skip, W_lin2):
    raise NotImplementedError("write your pallas kernel here")


# speedup vs baseline: 2.9786x; 2.9786x over previous
import math

import jax
import jax.numpy as jnp
from jax import lax
from jax.experimental import pallas as pl
from jax.experimental.pallas import tpu as pltpu
from jax.experimental.pallas import tpu_sc as plsc

N = 10000
E = 160000
A = 10
F = 16
S = 4
R = 8
O = 16

NC = 2
NS = 16
NW = NC * NS

CHUNK = 128
NCHUNK = E // CHUNK
W_BASE = NCHUNK // NW
W_REM = NCHUNK % NW

ROWS_PER_TILE = N // NS

SCALE_EDGE = 1.0 / math.sqrt(float(R * F * S * O))
SCALE_NODE = 1.0 / math.sqrt(float(O * A * O))

def _sc_mesh():
    return plsc.VectorSubcoreMesh(
        core_axis_name="c", subcore_axis_name="s", num_cores=NC, num_subcores=NS
    )


def _gather_body(nf_hbm, idx_hbm, out_hbm, idx_v, rows_v, sem):
    c = lax.axis_index("c")
    s = lax.axis_index("s")
    wid = s * NC + c
    start = wid * W_BASE + jnp.minimum(wid, W_REM)
    cnt = W_BASE + jnp.where(wid < W_REM, 1, 0)

    def step(j, carry):
        ch = start + j
        pltpu.sync_copy(idx_hbm.at[ch], idx_v)
        pltpu.async_copy(nf_hbm.at[idx_v], rows_v, sem).wait()
        pltpu.sync_copy(rows_v, out_hbm.at[pl.ds(ch * CHUNK, CHUNK)])
        return carry

    lax.fori_loop(0, cnt, step, 0)


def _sc_gather(node_feats, sender_chunks):
    return pl.kernel(
        _gather_body,
        out_type=jax.ShapeDtypeStruct((E, F), jnp.float32),
        mesh=_sc_mesh(),
        compiler_params=pltpu.CompilerParams(use_tc_tiling_on_sc=False),
        scratch_types=[
            pltpu.VMEM((CHUNK,), jnp.int32),
            pltpu.VMEM((CHUNK, F), jnp.float32),
            pltpu.SemaphoreType.DMA,
        ],
    )(node_feats, sender_chunks)


def _scatter_body(mji_hbm, idx_hbm, out_hbm, idx_v, rows_v, zbuf, acc, sem):
    c = lax.axis_index("c")
    s = lax.axis_index("s")
    wid = s * NC + c
    start = wid * W_BASE + jnp.minimum(wid, W_REM)
    cnt = W_BASE + jnp.where(wid < W_REM, 1, 0)

    def zstep(i, carry):
        zbuf[i] = jnp.zeros((F,), jnp.float32)
        return carry

    lax.fori_loop(0, ROWS_PER_TILE, zstep, 0)
    pltpu.sync_copy(zbuf, acc.at[pl.ds(s * ROWS_PER_TILE, ROWS_PER_TILE)])
    plsc.subcore_barrier()

    def step(j, carry):
        ch = start + j
        pltpu.sync_copy(idx_hbm.at[ch], idx_v)
        pltpu.sync_copy(mji_hbm.at[pl.ds(ch * CHUNK, CHUNK)], rows_v)
        pltpu.sync_copy(rows_v, acc.at[idx_v], add=True)
        return carry

    lax.fori_loop(0, cnt, step, 0)
    plsc.subcore_barrier()

    pltpu.sync_copy(
        acc.at[pl.ds(s * ROWS_PER_TILE, ROWS_PER_TILE)],
        out_hbm.at[c, pl.ds(s * ROWS_PER_TILE, ROWS_PER_TILE)],
    )


def _sc_scatter(mji, recv_chunks):
    return pl.kernel(
        _scatter_body,
        out_type=jax.ShapeDtypeStruct((NC, N, O), jnp.float32),
        mesh=_sc_mesh(),
        compiler_params=pltpu.CompilerParams(use_tc_tiling_on_sc=False),
        scratch_types=[
            pltpu.VMEM((CHUNK,), jnp.int32),
            pltpu.VMEM((CHUNK, O), jnp.float32),
            pltpu.VMEM((ROWS_PER_TILE, O), jnp.float32),
            pltpu.VMEM_SHARED((N, O), jnp.float32),
            pltpu.SemaphoreType.DMA,
        ],
    )(mji, recv_chunks)


EB = 2000


def _edge_body(xs_ref, ef_ref, ea_ref, wcat_ref, wlin1_ref, out_ref):
    xs = xs_ref[...]
    ef = ef_ref[...]
    ea = ea_ref[...]

    jj = lax.broadcasted_iota(jnp.int32, (R, R * F), 1)
    rr = lax.broadcasted_iota(jnp.int32, (R, R * F), 0)
    rep8 = (jj // F == rr).astype(jnp.float32)
    jj2 = lax.broadcasted_iota(jnp.int32, (F, R * F), 1)
    ff = lax.broadcasted_iota(jnp.int32, (F, R * F), 0)
    til16 = (jj2 % F == ff).astype(jnp.float32)

    v = jnp.dot(ef, rep8, preferred_element_type=jnp.float32) * jnp.dot(
        xs, til16, preferred_element_type=jnp.float32
    )
    p = jnp.dot(v, wcat_ref[...], preferred_element_type=jnp.float32)
    acc = ea[:, 0:1] * p[:, 0:O]
    for s in range(1, S):
        acc = acc + ea[:, s : s + 1] * p[:, s * O : (s + 1) * O]
    out_ref[...] = (
        jnp.dot(acc, wlin1_ref[...], preferred_element_type=jnp.float32) * SCALE_EDGE
    )


def _tc_edge(xs, edge_attrs, edge_feats, wcat, w_lin1):
    grid = (E // EB,)
    return pl.pallas_call(
        _edge_body,
        grid=grid,
        in_specs=[
            pl.BlockSpec((EB, F), lambda i: (i, 0)),
            pl.BlockSpec((EB, R), lambda i: (i, 0)),
            pl.BlockSpec((EB, S), lambda i: (i, 0)),
            pl.BlockSpec((R * F, S * O), lambda i: (0, 0)),
            pl.BlockSpec((O, O), lambda i: (0, 0)),
        ],
        out_specs=pl.BlockSpec((EB, O), lambda i: (i, 0)),
        out_shape=jax.ShapeDtypeStruct((E, O), jnp.float32),
    )(xs, edge_feats, edge_attrs, wcat, w_lin1)


NB = 2000


def _node_body(p_ref, na_ref, wsk_ref, wlin2_ref, out_ref):
    m = p_ref[0] + p_ref[1]
    na = na_ref[...]

    jj = lax.broadcasted_iota(jnp.int32, (O, O * A), 1)
    ff = lax.broadcasted_iota(jnp.int32, (O, O * A), 0)
    rep16 = (jj // A == ff).astype(jnp.float32)
    jj2 = lax.broadcasted_iota(jnp.int32, (A, O * A), 1)
    aa = lax.broadcasted_iota(jnp.int32, (A, O * A), 0)
    til10 = (jj2 % A == aa).astype(jnp.float32)

    v2 = jnp.dot(m, rep16, preferred_element_type=jnp.float32) * jnp.dot(
        na, til10, preferred_element_type=jnp.float32
    )
    x1 = jnp.dot(v2, wsk_ref[...], preferred_element_type=jnp.float32)
    x2 = jnp.dot(x1, wlin2_ref[...], preferred_element_type=jnp.float32)
    out_ref[...] = m + x2 * SCALE_NODE


def _tc_node(partials, node_attrs, wsk, w_lin2):
    grid = (N // NB,)
    return pl.pallas_call(
        _node_body,
        grid=grid,
        in_specs=[
            pl.BlockSpec((NC, NB, O), lambda i: (0, i, 0)),
            pl.BlockSpec((NB, A), lambda i: (i, 0)),
            pl.BlockSpec((O * A, O), lambda i: (0, 0)),
            pl.BlockSpec((O, O), lambda i: (0, 0)),
        ],
        out_specs=pl.BlockSpec((NB, O), lambda i: (i, 0)),
        out_shape=jax.ShapeDtypeStruct((N, O), jnp.float32),
    )(partials, node_attrs, wsk, w_lin2)


def kernel(node_attrs, node_feats, edge_attrs, edge_feats, edge_index,
           W_tpw, W_lin1, W_skip, W_lin2):
    sender_chunks = edge_index[0].reshape(NCHUNK, CHUNK)
    recv_chunks = edge_index[1].reshape(NCHUNK, CHUNK)
    wcat = W_tpw.reshape(R * F, S * O)
    wsk = W_skip.reshape(O * A, O)

    xs = _sc_gather(node_feats, sender_chunks)
    mji = _tc_edge(xs, edge_attrs, edge_feats, wcat, W_lin1)
    partials = _sc_scatter(mji, recv_chunks)
    out = _tc_node(partials, node_attrs, wsk, W_lin2)
    return out
